# Initial kernel scaffold; baseline (speedup 1.0000x reference)
import jax
import jax.numpy as jnp
from jax.experimental import pallas as pl


def kernel(i, j, l, label, n_emb, l_emb, w):
    raise NotImplementedError("write your pallas kernel here")


# speedup vs baseline: 2.2942x; 2.2942x over previous
import functools

import jax
import jax.numpy as jnp
from jax import lax
from jax.experimental import pallas as pl
from jax.experimental.pallas import tpu as pltpu
from jax.experimental.pallas import tpu_sc as plsc

NUM_NODES = 100000
NODE_DIM = 128
LAYER_DIM = 64
NUM_LAYER = 8
BATCH = 4096

NC = 2
NS = 16
NW = NC * NS

GB = 2 * BATCH
N_PER_W = GB // NW
CHUNK = 128
N_CHUNKS = N_PER_W // CHUNK


def _sc_gather_body(idx_hbm, nemb_hbm, out_g_hbm, idx_v, rows_v, sem):
  wid = lax.axis_index("s") * NC + lax.axis_index("c")
  nb = wid * N_PER_W
  for c in range(N_CHUNKS):
    pltpu.sync_copy(idx_hbm.at[N_CHUNKS * wid + c], idx_v.at[c])
  copies = []
  for c in range(N_CHUNKS):
    copies.append(pltpu.async_copy(
        nemb_hbm.at[idx_v.at[c]], rows_v.at[pl.ds(c * CHUNK, CHUNK)], sem))
  for cp in copies:
    cp.wait()
  pltpu.sync_copy(rows_v, out_g_hbm.at[pl.ds(nb, N_PER_W)])


@functools.cache
def _sc_gather():
  return pl.kernel(
      _sc_gather_body,
      out_type=jax.ShapeDtypeStruct((GB, NODE_DIM), jnp.float32),
      mesh=plsc.VectorSubcoreMesh(
          core_axis_name="c", subcore_axis_name="s",
          num_cores=NC, num_subcores=NS),
      scratch_types=[
          pltpu.VMEM((N_CHUNKS, CHUNK), jnp.int32),
          pltpu.VMEM((N_PER_W, NODE_DIM), jnp.float32),
          pltpu.SemaphoreType.DMA,
      ],
  )


def _tc_body(g_ref, l_ref, label_ref, lemb_ref, w_ref, out_ref):
  g = g_ref[...]
  w = w_ref[...]
  pq = jnp.dot(g, w, preferred_element_type=jnp.float32)
  p = pq[:BATCH]
  q = pq[BATCH:]
  li = l_ref[...]
  oh = (lax.broadcasted_iota(jnp.int32, (BATCH, NUM_LAYER), 1)
        == li).astype(jnp.float32)
  lt = jnp.dot(oh, lemb_ref[...], preferred_element_type=jnp.float32)
  s = jnp.sum((lt + p) * (lt + q))
  z = label_ref[...] * s
  ls = jnp.minimum(z, 0.0) - jnp.log1p(jnp.exp(-jnp.abs(z)))
  out_ref[...] = (-jnp.sum(ls)).reshape(1, 1)


def kernel(i, j, l, label, n_emb, l_emb, w):
  idx = jnp.concatenate([i, j]).astype(jnp.int32).reshape(NW * N_CHUNKS, CHUNK)
  g = _sc_gather()(idx, n_emb)
  out = pl.pallas_call(
      _tc_body,
      out_shape=jax.ShapeDtypeStruct((1, 1), jnp.float32),
  )(g, l.astype(jnp.int32).reshape(BATCH, 1), label.reshape(BATCH, 1),
    l_emb, w)
  return out[0, 0]


# speedup vs baseline: 2.3100x; 1.0069x over previous
import functools

import jax
import jax.numpy as jnp
from jax import lax
from jax.experimental import pallas as pl
from jax.experimental.pallas import tpu as pltpu
from jax.experimental.pallas import tpu_sc as plsc

NUM_NODES = 100000
NODE_DIM = 128
LAYER_DIM = 64
NUM_LAYER = 8
BATCH = 4096

NC = 2
NS = 16
NW = NC * NS

GB = 2 * BATCH
N_PER_W = GB // NW
CHUNK = 128
N_CHUNKS = N_PER_W // CHUNK


def _sc_gather_body(i_hbm, j_hbm, nemb_hbm, out_g_hbm, idx_v, rows_v,
                    gsem, wsem):
  wid = lax.axis_index("s") * NC + lax.axis_index("c")
  base = wid * CHUNK
  pltpu.sync_copy(i_hbm.at[pl.ds(base, CHUNK)], idx_v.at[0])
  pltpu.sync_copy(j_hbm.at[pl.ds(base, CHUNK)], idx_v.at[1])
  cp0 = pltpu.async_copy(nemb_hbm.at[idx_v.at[0]],
                         rows_v.at[pl.ds(0, CHUNK)], gsem)
  cp1 = pltpu.async_copy(nemb_hbm.at[idx_v.at[1]],
                         rows_v.at[pl.ds(CHUNK, CHUNK)], gsem)
  cp0.wait()
  w0 = pltpu.async_copy(rows_v.at[pl.ds(0, CHUNK)],
                        out_g_hbm.at[pl.ds(base, CHUNK)], wsem)
  cp1.wait()
  w1 = pltpu.async_copy(rows_v.at[pl.ds(CHUNK, CHUNK)],
                        out_g_hbm.at[pl.ds(BATCH + base, CHUNK)], wsem)
  w0.wait()
  w1.wait()


@functools.cache
def _sc_gather():
  return pl.kernel(
      _sc_gather_body,
      out_type=jax.ShapeDtypeStruct((GB, NODE_DIM), jnp.float32),
      mesh=plsc.VectorSubcoreMesh(
          core_axis_name="c", subcore_axis_name="s",
          num_cores=NC, num_subcores=NS),
      scratch_types=[
          pltpu.VMEM((N_CHUNKS, CHUNK), jnp.int32),
          pltpu.VMEM((N_PER_W, NODE_DIM), jnp.float32),
          pltpu.SemaphoreType.DMA,
          pltpu.SemaphoreType.DMA,
      ],
  )


def _tc_body(g_ref, l_ref, label_ref, lemb_ref, w_ref, out_ref):
  g = g_ref[...]
  w = w_ref[...]
  pq = jnp.dot(g, w, preferred_element_type=jnp.float32)
  p = pq[:BATCH]
  q = pq[BATCH:]
  li = l_ref[...]
  oh = (lax.broadcasted_iota(jnp.int32, (BATCH, NUM_LAYER), 1)
        == li).astype(jnp.float32)
  lt = jnp.dot(oh, lemb_ref[...], preferred_element_type=jnp.float32)
  s = jnp.sum((lt + p) * (lt + q))
  z = label_ref[...] * s
  ls = jnp.minimum(z, 0.0) - jnp.log1p(jnp.exp(-jnp.abs(z)))
  out_ref[...] = (-jnp.sum(ls)).reshape(1, 1)


def kernel(i, j, l, label, n_emb, l_emb, w):
  g = _sc_gather()(i.astype(jnp.int32), j.astype(jnp.int32), n_emb)
  out = pl.pallas_call(
      _tc_body,
      out_shape=jax.ShapeDtypeStruct((1, 1), jnp.float32),
  )(g, l.astype(jnp.int32).reshape(BATCH, 1), label.reshape(BATCH, 1),
    l_emb, w)
  return out[0, 0]
